# baseline (speedup 1.0000x reference)
import functools

import jax
import jax.numpy as jnp
from jax.experimental import pallas as pl
from jax.experimental.pallas import tpu as pltpu

_BM_A = 200
_BM_B = 5000
_EPS = 1e-5


def _fused_body(inp_ref, adj_ref, gamma_ref, beta_ref, w1_ref, w2_ref,
                b_ref, out_ref, inp_bf_ref, agg_ref, stats_ref, *,
                nb_a, n_rows):
    i = pl.program_id(0)

    @pl.when(i == 0)
    def _init():
        stats_ref[...] = jnp.zeros_like(stats_ref)
        inp_bf_ref[...] = inp_ref[...].astype(jnp.bfloat16)

    @pl.when(i < nb_a)
    def _phase_a():
        a = adj_ref[...].astype(jnp.bfloat16)
        o = jnp.dot(a, inp_bf_ref[...], preferred_element_type=jnp.float32)
        agg_ref[pl.ds(i * _BM_A, _BM_A), :] = o
        xin = inp_ref[pl.ds(i * _BM_A, _BM_A), :]
        stats_ref[0:1, :] = stats_ref[0:1, :] + jnp.sum(xin, axis=0, keepdims=True)
        stats_ref[1:2, :] = stats_ref[1:2, :] + jnp.sum(xin * xin, axis=0, keepdims=True)
        stats_ref[2:3, :] = stats_ref[2:3, :] + jnp.sum(o, axis=0, keepdims=True)
        stats_ref[3:4, :] = stats_ref[3:4, :] + jnp.sum(o * o, axis=0, keepdims=True)

    @pl.when(i >= nb_a)
    def _phase_b():
        j = i - nb_a
        inv_n = 1.0 / n_rows
        mean1 = stats_ref[0:1, :] * inv_n
        var1 = stats_ref[1:2, :] * inv_n - mean1 * mean1
        mean2 = stats_ref[2:3, :] * inv_n
        var2 = stats_ref[3:4, :] * inv_n - mean2 * mean2
        scale1 = gamma_ref[0:1, :] * jax.lax.rsqrt(var1 + _EPS)
        scale2 = gamma_ref[1:2, :] * jax.lax.rsqrt(var2 + _EPS)
        xin = inp_ref[pl.ds(j * _BM_B, _BM_B), :]
        xagg = agg_ref[pl.ds(j * _BM_B, _BM_B), :]
        h1 = ((xin - mean1) * scale1 + beta_ref[0:1, :]).astype(jnp.bfloat16)
        h2 = ((xagg - mean2) * scale2 + beta_ref[1:2, :]).astype(jnp.bfloat16)
        dims = (((1,), (1,)), ((), ()))
        d1 = jax.lax.dot_general(h1, w1_ref[...], dims,
                                 preferred_element_type=jnp.float32)
        d2 = jax.lax.dot_general(h2, w2_ref[...], dims,
                                 preferred_element_type=jnp.float32)
        out_ref[...] = d1 + d2 + b_ref[...]


def kernel(input, adj, gamma, beta, W, b):
    n, d = input.shape
    nb_a = n // _BM_A
    nb_b = n // _BM_B

    gamma2 = gamma.reshape(2, d)
    beta2 = beta.reshape(2, d)
    w1 = W[:, :d].astype(jnp.bfloat16)
    w2 = W[:, d:].astype(jnp.bfloat16)
    b_row = b.reshape(1, d)

    last_a = nb_a - 1
    out = pl.pallas_call(
        functools.partial(_fused_body, nb_a=nb_a, n_rows=float(n)),
        grid=(nb_a + nb_b,),
        in_specs=[
            pl.BlockSpec((n, d), lambda i: (0, 0)),
            pl.BlockSpec((_BM_A, n), lambda i: (jnp.minimum(i, last_a), 0)),
            pl.BlockSpec((2, d), lambda i: (0, 0)),
            pl.BlockSpec((2, d), lambda i: (0, 0)),
            pl.BlockSpec((d, d), lambda i: (0, 0)),
            pl.BlockSpec((d, d), lambda i: (0, 0)),
            pl.BlockSpec((1, d), lambda i: (0, 0)),
        ],
        out_specs=pl.BlockSpec(
            (_BM_B, d), lambda i: (jnp.maximum(i - nb_a, 0), 0)),
        out_shape=jax.ShapeDtypeStruct((n, d), jnp.float32),
        scratch_shapes=[
            pltpu.VMEM((n, d), jnp.bfloat16),
            pltpu.VMEM((n, d), jnp.float32),
            pltpu.VMEM((8, d), jnp.float32),
        ],
    )(input, adj, gamma2, beta2, w1, w2, b_row)
    return out


# speedup vs baseline: 1.0117x; 1.0117x over previous
import functools

import jax
import jax.numpy as jnp
from jax.experimental import pallas as pl
from jax.experimental.pallas import tpu as pltpu

_BM_A = 400
_BM_B = 5000
_EPS = 1e-5


def _fused_body(inp_ref, adj_ref, gamma_ref, beta_ref, w1_ref, w2_ref,
                b_ref, out_ref, inp_bf_ref, agg_ref, stats_ref, *,
                nb_a, n_rows):
    i = pl.program_id(0)

    @pl.when(i == 0)
    def _init():
        stats_ref[...] = jnp.zeros_like(stats_ref)
        inp_bf_ref[...] = inp_ref[...].astype(jnp.bfloat16)

    @pl.when(i < nb_a)
    def _phase_a():
        a = adj_ref[...].astype(jnp.bfloat16)
        o = jnp.dot(a, inp_bf_ref[...], preferred_element_type=jnp.float32)
        agg_ref[pl.ds(i * _BM_A, _BM_A), :] = o
        xin = inp_ref[pl.ds(i * _BM_A, _BM_A), :]
        stats_ref[0:1, :] = stats_ref[0:1, :] + jnp.sum(xin, axis=0, keepdims=True)
        stats_ref[1:2, :] = stats_ref[1:2, :] + jnp.sum(xin * xin, axis=0, keepdims=True)
        stats_ref[2:3, :] = stats_ref[2:3, :] + jnp.sum(o, axis=0, keepdims=True)
        stats_ref[3:4, :] = stats_ref[3:4, :] + jnp.sum(o * o, axis=0, keepdims=True)

    @pl.when(i >= nb_a)
    def _phase_b():
        j = i - nb_a
        inv_n = 1.0 / n_rows
        mean1 = stats_ref[0:1, :] * inv_n
        var1 = stats_ref[1:2, :] * inv_n - mean1 * mean1
        mean2 = stats_ref[2:3, :] * inv_n
        var2 = stats_ref[3:4, :] * inv_n - mean2 * mean2
        scale1 = gamma_ref[0:1, :] * jax.lax.rsqrt(var1 + _EPS)
        scale2 = gamma_ref[1:2, :] * jax.lax.rsqrt(var2 + _EPS)
        xin = inp_ref[pl.ds(j * _BM_B, _BM_B), :]
        xagg = agg_ref[pl.ds(j * _BM_B, _BM_B), :]
        h1 = ((xin - mean1) * scale1 + beta_ref[0:1, :]).astype(jnp.bfloat16)
        h2 = ((xagg - mean2) * scale2 + beta_ref[1:2, :]).astype(jnp.bfloat16)
        dims = (((1,), (1,)), ((), ()))
        d1 = jax.lax.dot_general(h1, w1_ref[...], dims,
                                 preferred_element_type=jnp.float32)
        d2 = jax.lax.dot_general(h2, w2_ref[...], dims,
                                 preferred_element_type=jnp.float32)
        out_ref[...] = d1 + d2 + b_ref[...]


def kernel(input, adj, gamma, beta, W, b):
    n, d = input.shape
    nb_a = n // _BM_A
    nb_b = n // _BM_B

    gamma2 = gamma.reshape(2, d)
    beta2 = beta.reshape(2, d)
    w1 = W[:, :d].astype(jnp.bfloat16)
    w2 = W[:, d:].astype(jnp.bfloat16)
    b_row = b.reshape(1, d)

    last_a = nb_a - 1
    out = pl.pallas_call(
        functools.partial(_fused_body, nb_a=nb_a, n_rows=float(n)),
        grid=(nb_a + nb_b,),
        in_specs=[
            pl.BlockSpec((n, d), lambda i: (0, 0)),
            pl.BlockSpec((_BM_A, n), lambda i: (jnp.minimum(i, last_a), 0)),
            pl.BlockSpec((2, d), lambda i: (0, 0)),
            pl.BlockSpec((2, d), lambda i: (0, 0)),
            pl.BlockSpec((d, d), lambda i: (0, 0)),
            pl.BlockSpec((d, d), lambda i: (0, 0)),
            pl.BlockSpec((1, d), lambda i: (0, 0)),
        ],
        out_specs=pl.BlockSpec(
            (_BM_B, d), lambda i: (jnp.maximum(i - nb_a, 0), 0)),
        out_shape=jax.ShapeDtypeStruct((n, d), jnp.float32),
        scratch_shapes=[
            pltpu.VMEM((n, d), jnp.bfloat16),
            pltpu.VMEM((n, d), jnp.float32),
            pltpu.VMEM((8, d), jnp.float32),
        ],
    )(input, adj, gamma2, beta2, w1, w2, b_row)
    return out


# speedup vs baseline: 1.0320x; 1.0200x over previous
import functools

import jax
import jax.numpy as jnp
from jax.experimental import pallas as pl
from jax.experimental.pallas import tpu as pltpu

_BM = 80
_NBUF = 5
_EPS = 1e-5


def _body(inp_ref, gamma_ref, beta_ref, w1_ref, w2_ref, b_ref, adj_hbm,
          out_hbm, inp_bf_ref, agg_ref, stats_ref, adj_buf, out_buf,
          sems, osems, *, n, d):
    ns = n // _BM

    for k in range(_NBUF):
        pltpu.make_async_copy(adj_hbm.at[pl.ds(k * _BM, _BM), :],
                              adj_buf.at[k], sems.at[k]).start()

    inp_bf_ref[...] = inp_ref[...].astype(jnp.bfloat16)
    stats_ref[...] = jnp.zeros_like(stats_ref)

    def _round(r, carry):
        for k in range(_NBUF):
            s = r * _NBUF + k
            pltpu.make_async_copy(adj_hbm.at[pl.ds(s * _BM, _BM), :],
                                  adj_buf.at[k], sems.at[k]).wait()
            a = adj_buf[k].astype(jnp.bfloat16)
            o = jnp.dot(a, inp_bf_ref[...],
                        preferred_element_type=jnp.float32)
            agg_ref[pl.ds(s * _BM, _BM), :] = o
            xin = inp_ref[pl.ds(s * _BM, _BM), :]
            stats_ref[0:1, :] = stats_ref[0:1, :] + jnp.sum(
                xin, axis=0, keepdims=True)
            stats_ref[1:2, :] = stats_ref[1:2, :] + jnp.sum(
                xin * xin, axis=0, keepdims=True)
            stats_ref[2:3, :] = stats_ref[2:3, :] + jnp.sum(
                o, axis=0, keepdims=True)
            stats_ref[3:4, :] = stats_ref[3:4, :] + jnp.sum(
                o * o, axis=0, keepdims=True)

            @pl.when(s + _NBUF < ns)
            def _prefetch():
                pltpu.make_async_copy(
                    adj_hbm.at[pl.ds((s + _NBUF) * _BM, _BM), :],
                    adj_buf.at[k], sems.at[k]).start()
        return carry

    jax.lax.fori_loop(0, ns // _NBUF, _round, 0)

    inv_n = 1.0 / n
    mean1 = stats_ref[0:1, :] * inv_n
    var1 = stats_ref[1:2, :] * inv_n - mean1 * mean1
    mean2 = stats_ref[2:3, :] * inv_n
    var2 = stats_ref[3:4, :] * inv_n - mean2 * mean2
    scale1 = gamma_ref[0:1, :] * jax.lax.rsqrt(var1 + _EPS)
    scale2 = gamma_ref[1:2, :] * jax.lax.rsqrt(var2 + _EPS)
    dims = (((1,), (1,)), ((), ()))
    half = n // 2
    for c in range(2):
        xin = inp_ref[pl.ds(c * half, half), :]
        xagg = agg_ref[pl.ds(c * half, half), :]
        h1 = ((xin - mean1) * scale1 + beta_ref[0:1, :]).astype(jnp.bfloat16)
        h2 = ((xagg - mean2) * scale2 + beta_ref[1:2, :]).astype(jnp.bfloat16)
        d1 = jax.lax.dot_general(h1, w1_ref[...], dims,
                                 preferred_element_type=jnp.float32)
        d2 = jax.lax.dot_general(h2, w2_ref[...], dims,
                                 preferred_element_type=jnp.float32)
        out_buf[c] = d1 + d2 + b_ref[...]
        pltpu.make_async_copy(out_buf.at[c],
                              out_hbm.at[pl.ds(c * half, half), :],
                              osems.at[c]).start()
    for c in range(2):
        pltpu.make_async_copy(out_buf.at[c],
                              out_hbm.at[pl.ds(c * half, half), :],
                              osems.at[c]).wait()


def kernel(input, adj, gamma, beta, W, b):
    n, d = input.shape

    gamma2 = gamma.reshape(2, d)
    beta2 = beta.reshape(2, d)
    w1 = W[:, :d].astype(jnp.bfloat16)
    w2 = W[:, d:].astype(jnp.bfloat16)
    b_row = b.reshape(1, d)

    out = pl.pallas_call(
        functools.partial(_body, n=n, d=d),
        in_specs=[
            pl.BlockSpec((n, d), lambda: (0, 0)),
            pl.BlockSpec((2, d), lambda: (0, 0)),
            pl.BlockSpec((2, d), lambda: (0, 0)),
            pl.BlockSpec((d, d), lambda: (0, 0)),
            pl.BlockSpec((d, d), lambda: (0, 0)),
            pl.BlockSpec((1, d), lambda: (0, 0)),
            pl.BlockSpec(memory_space=pl.ANY),
        ],
        out_specs=pl.BlockSpec(memory_space=pl.ANY),
        out_shape=jax.ShapeDtypeStruct((n, d), jnp.float32),
        scratch_shapes=[
            pltpu.VMEM((n, d), jnp.bfloat16),
            pltpu.VMEM((n, d), jnp.float32),
            pltpu.VMEM((8, d), jnp.float32),
            pltpu.VMEM((_NBUF, _BM, n), jnp.float32),
            pltpu.VMEM((2, n // 2, d), jnp.float32),
            pltpu.SemaphoreType.DMA((_NBUF,)),
            pltpu.SemaphoreType.DMA((2,)),
        ],
    )(input, gamma2, beta2, w1, w2, b_row, adj)
    return out
